# baseline (speedup 1.0000x reference)
import functools

import jax
import jax.numpy as jnp
from jax import lax
from jax.experimental import pallas as pl
from jax.experimental.pallas import tpu as pltpu
from jax.experimental.pallas import tpu_sc as plsc

_NUM_LABELS = 1000
_EMBED = 64
_LANES = 128
_NC = 2
_NS = 16
_NW = _NC * _NS
_CHUNK = 128


def _make_gather(n_idx):
    assert n_idx % (_NW * _CHUNK) == 0
    per_w = n_idx // _NW
    n_chunks = per_w // _CHUNK
    mesh = plsc.VectorSubcoreMesh(
        core_axis_name="c", subcore_axis_name="s",
        num_cores=_NC, num_subcores=_NS,
    )

    @functools.partial(
        pl.kernel,
        out_type=jax.ShapeDtypeStruct((n_idx, _LANES), jnp.float32),
        mesh=mesh,
        scratch_types=[
            pltpu.VMEM((per_w,), jnp.int32),
            pltpu.VMEM((per_w, _EMBED), jnp.float32),
            pltpu.SemaphoreType.DMA,
        ],
        compiler_params=pltpu.CompilerParams(use_tc_tiling_on_sc=False),
    )
    def gather(table_hbm, idx_hbm, out_hbm, idx_v, rows_v, sem):
        wid = lax.axis_index("s") * _NC + lax.axis_index("c")
        pltpu.sync_copy(idx_hbm.at[pl.ds(wid * per_w, per_w)], idx_v)
        copies = [
            pltpu.async_copy(
                table_hbm.at[idx_v.at[pl.ds(c * _CHUNK, _CHUNK)]],
                rows_v.at[pl.ds(c * _CHUNK, _CHUNK)],
                sem,
            )
            for c in range(n_chunks)
        ]
        for cp in copies:
            cp.wait()
        pltpu.sync_copy(
            rows_v,
            out_hbm.at[pl.ds(wid * per_w, per_w), pl.ds(0, _EMBED)],
        )

    return gather


def _finalize_body(rows_ref, b_ref, out_ref):
    x = rows_ref[...]
    blk = out_ref.shape[0]
    y = x.reshape(blk, out_ref.shape[1], _LANES)[:, :, :_EMBED]
    out_ref[...] = y + b_ref[0][None, None, :]


def _finalize(rows, b2, B, S):
    grid = 16
    blk = B // grid
    return pl.pallas_call(
        _finalize_body,
        grid=(grid,),
        in_specs=[
            pl.BlockSpec((blk * S, _LANES), lambda i: (i, 0)),
            pl.BlockSpec((1, _EMBED), lambda i: (0, 0)),
        ],
        out_specs=pl.BlockSpec((blk, S, _EMBED), lambda i: (i, 0, 0)),
        out_shape=jax.ShapeDtypeStruct((B, S, _EMBED), jnp.float32),
    )(rows, b2)


def kernel(X, W, b):
    B, S, _ = X.shape
    n_idx = B * S
    table = W.T
    idx = X.reshape(n_idx).astype(jnp.int32)
    rows = _make_gather(n_idx)(table, idx)
    return _finalize(rows, b.reshape(1, _EMBED), B, S)


# speedup vs baseline: 1.0352x; 1.0352x over previous
import functools

import jax
import jax.numpy as jnp
from jax import lax
from jax.experimental import pallas as pl
from jax.experimental.pallas import tpu as pltpu
from jax.experimental.pallas import tpu_sc as plsc

_NUM_LABELS = 1000
_EMBED = 64
_NC = 2
_NS = 16
_NW = _NC * _NS
_CHUNK = 128


def _make_gather(B, S):
    n_idx = B * S
    assert n_idx % (_NW * _CHUNK) == 0 and B % _NW == 0
    per_w = n_idx // _NW
    n_chunks = per_w // _CHUNK
    b_per_w = B // _NW
    mesh = plsc.VectorSubcoreMesh(
        core_axis_name="c", subcore_axis_name="s",
        num_cores=_NC, num_subcores=_NS,
    )

    @functools.partial(
        pl.kernel,
        out_type=jax.ShapeDtypeStruct((B, S, _EMBED), jnp.float32),
        mesh=mesh,
        scratch_types=[
            pltpu.VMEM((per_w,), jnp.int32),
            pltpu.VMEM((per_w, _EMBED), jnp.float32),
            pltpu.SemaphoreType.DMA,
            pltpu.SemaphoreType.DMA,
        ],
        compiler_params=pltpu.CompilerParams(use_tc_tiling_on_sc=False),
    )
    def gather(table_hbm, idx_hbm, out_hbm, idx_v, rows_v, gsem, wsem):
        wid = lax.axis_index("s") * _NC + lax.axis_index("c")
        pltpu.sync_copy(idx_hbm.at[pl.ds(wid * per_w, per_w)], idx_v)
        copies = [
            pltpu.async_copy(
                table_hbm.at[idx_v.at[pl.ds(c * _CHUNK, _CHUNK)]],
                rows_v.at[pl.ds(c * _CHUNK, _CHUNK)],
                gsem,
            )
            for c in range(n_chunks)
        ]
        writes = []
        fired = 0
        for c in range(n_chunks):
            copies[c].wait()
            covered = ((c + 1) * _CHUNK) // S
            for p in range(fired, min(covered, b_per_w)):
                writes.append(
                    pltpu.async_copy(
                        rows_v.at[pl.ds(p * S, S)],
                        out_hbm.at[wid * b_per_w + p],
                        wsem,
                    )
                )
            fired = min(covered, b_per_w)
        for cp in writes:
            cp.wait()

    return gather


def kernel(X, W, b):
    B, S, _ = X.shape
    table = W.T + b[None, :]
    idx = X.reshape(B * S).astype(jnp.int32)
    return _make_gather(B, S)(table, idx)


# speedup vs baseline: 1.0683x; 1.0320x over previous
import functools

import jax
import jax.numpy as jnp
from jax import lax
from jax.experimental import pallas as pl
from jax.experimental.pallas import tpu as pltpu
from jax.experimental.pallas import tpu_sc as plsc

_NUM_LABELS = 1000
_EMBED = 64
_NC = 2
_NS = 16
_NW = _NC * _NS
_CHUNK = 128


def _make_gather(B, S):
    n_idx = B * S
    assert n_idx % (_NW * _CHUNK) == 0 and B % _NW == 0
    per_w = n_idx // _NW
    n_chunks = per_w // _CHUNK
    b_per_w = B // _NW
    mesh = plsc.VectorSubcoreMesh(
        core_axis_name="c", subcore_axis_name="s",
        num_cores=_NC, num_subcores=_NS,
    )

    @functools.partial(
        pl.kernel,
        out_type=jax.ShapeDtypeStruct((B, S, _EMBED), jnp.float32),
        mesh=mesh,
        scratch_types=[
            pltpu.VMEM((per_w,), jnp.int32),
            pltpu.VMEM((per_w, _EMBED), jnp.float32),
            pltpu.SemaphoreType.DMA,
            pltpu.SemaphoreType.DMA,
        ],
        compiler_params=pltpu.CompilerParams(use_tc_tiling_on_sc=False),
    )
    def gather(table_hbm, idx_hbm, out_hbm, idx_v, rows_v, gsem, wsem):
        wid = lax.axis_index("s") * _NC + lax.axis_index("c")
        pltpu.sync_copy(idx_hbm.at[pl.ds(wid * per_w, per_w)], idx_v)
        pltpu.async_copy(table_hbm.at[idx_v], rows_v, gsem).wait()
        writes = [
            pltpu.async_copy(
                rows_v.at[pl.ds(p * S, S)],
                out_hbm.at[wid * b_per_w + p],
                wsem,
            )
            for p in range(b_per_w)
        ]
        for cp in writes:
            cp.wait()

    return gather


def kernel(X, W, b):
    B, S, _ = X.shape
    table = W.T + b[None, :]
    idx = X.reshape(B * S).astype(jnp.int32)
    return _make_gather(B, S)(table, idx)
